# baseline (speedup 1.0000x reference)
import jax
import jax.numpy as jnp
import numpy as np
from jax import lax
from jax.experimental import pallas as pl
from jax.experimental.pallas import tpu as pltpu

T = 32768
D_IN = 4
D_HID = 16
D_FF = 32
E = 8
TT = 16384

_EYE_E = np.eye(E, dtype=np.float32)
_TILE16_T = np.tile(np.eye(D_HID, dtype=np.float32), (E, 1))
_GSUM = np.tile(np.eye(D_HID, dtype=np.float32), (1, E))
_S128_T = np.repeat(_EYE_E, D_HID, axis=0)
_PREFIX_T = np.tril(np.ones((E, E), dtype=np.float32), -1)

_NN = (((1,), (0,)), ((), ()))
_TN = (((0,), (0,)), ((), ()))
_TX = (((0,), (1,)), ((), ()))


def _moe_block(x_ref, We_ref, be_ref, Wg_ref, Wi_ref, bi_ref, Wo_ref, bo_ref,
               Wp_ref, bp_ref, tile16_ref, gsum_ref, s128_ref,
               prefix_ref, out_ref):
    _moe_half(x_ref, We_ref, be_ref, Wg_ref, Wi_ref, bi_ref, Wo_ref, bo_ref,
              Wp_ref, bp_ref, tile16_ref, gsum_ref, s128_ref, prefix_ref,
              out_ref, 0)
    _moe_half(x_ref, We_ref, be_ref, Wg_ref, Wi_ref, bi_ref, Wo_ref, bo_ref,
              Wp_ref, bp_ref, tile16_ref, gsum_ref, s128_ref, prefix_ref,
              out_ref, 1)


def _moe_half(x_ref, We_ref, be_ref, Wg_ref, Wi_ref, bi_ref, Wo_ref, bo_ref,
              Wp_ref, bp_ref, tile16_ref, gsum_ref, s128_ref, prefix_ref,
              out_ref, half):
    f32 = jnp.float32
    dot = lambda a, b, dn=_NN: lax.dot_general(a, b, dn,
                                               preferred_element_type=f32)
    HH = TT // 2
    x = x_ref[pl.ds(half * HH, HH), :]
    h = dot(We_ref[...], x, _TX) + be_ref[...]
    logits = dot(Wg_ref[...], h, _TN)
    m = jnp.max(logits, axis=0, keepdims=True)
    el = jnp.exp(logits - m)
    gate = 1.0 / jnp.sum(el, axis=0, keepdims=True)
    bf16 = jnp.bfloat16
    is_max = (logits >= m).astype(bf16)
    prior = dot(prefix_ref[...], is_max)
    onehot = jnp.where(prior == 0.0, is_max, jnp.zeros_like(is_max))
    onehot_f = onehot.astype(f32)
    mask = dot(s128_ref[...], onehot)
    h_exp = dot(tile16_ref[...], h) * mask
    wi_rows = Wi_ref[...].reshape(E * D_HID, D_FF)
    pre = dot(wi_rows, h_exp, _TN) + dot(bi_ref[...], onehot_f, _TN)
    mid = jax.nn.gelu(pre)
    wo_all = jnp.transpose(Wo_ref[...], (0, 2, 1)).reshape(E * D_HID, D_FF)
    o_all = dot(wo_all, mid)
    moe = (dot(gsum_ref[...], o_all * mask)
           + dot(bo_ref[...], onehot_f, _TN)) * gate
    out_t = dot(Wp_ref[...], moe, _TN) + bp_ref[...]
    out_ref[pl.ds(half * HH, HH), :] = out_t.T


def kernel(x, W_embed, b_embed, W_gate, Wi, bi, Wo, bo, W_proj, b_proj):
    full = lambda a: pl.BlockSpec(a.shape, lambda i: (0,) * a.ndim)
    args = (W_embed, b_embed.reshape(D_HID, 1), W_gate, Wi, bi, Wo, bo,
            W_proj, b_proj.reshape(D_IN, 1),
            jnp.asarray(_TILE16_T), jnp.asarray(_GSUM),
            jnp.asarray(_S128_T, dtype=jnp.bfloat16),
            jnp.asarray(_PREFIX_T, dtype=jnp.bfloat16))
    return pl.pallas_call(
        _moe_block,
        grid=(T // TT,),
        in_specs=[pl.BlockSpec((TT, D_IN), lambda i: (i, 0))]
                 + [full(a) for a in args],
        out_specs=pl.BlockSpec((TT, D_IN), lambda i: (i, 0)),
        out_shape=jax.ShapeDtypeStruct((T, D_IN), jnp.float32),
        compiler_params=pltpu.CompilerParams(
            dimension_semantics=("parallel",)),
    )(x, *args)


# speedup vs baseline: 1.0743x; 1.0743x over previous
import jax
import jax.numpy as jnp
import numpy as np
from jax import lax
from jax.experimental import pallas as pl
from jax.experimental.pallas import tpu as pltpu

T = 32768
D_IN = 4
D_HID = 16
D_FF = 32
E = 8
TT = 8192

_EYE_E = np.eye(E, dtype=np.float32)
_TILE4 = np.tile(np.eye(D_IN, dtype=np.float32), (E, 1))
_G4 = np.tile(np.eye(D_IN, dtype=np.float32), (1, E))
_S32 = np.repeat(_EYE_E, D_IN, axis=0)
_PREFIX_T = np.tril(np.ones((E, E), dtype=np.float32), -1)

_NN = (((1,), (0,)), ((), ()))
_TN = (((0,), (0,)), ((), ()))
_TX = (((0,), (1,)), ((), ()))
_NT = (((1,), (1,)), ((), ()))


def _dot(a, b, dn=_NN):
    return lax.dot_general(a, b, dn, preferred_element_type=jnp.float32)


def _moe_block(x_ref, We_ref, be_ref, Wg_ref, Wi_ref, bi_ref, Wo_ref, bo_ref,
               Wp_ref, bp_ref, tile4_ref, g4_ref, s32_ref, prefix_ref,
               out_ref):
    wi2 = jnp.transpose(Wi_ref[...], (1, 0, 2)).reshape(D_HID, E * D_FF)
    ci = _dot(We_ref[...], wi2)
    ci_rows = jnp.transpose(ci.reshape(D_IN, E, D_FF),
                            (1, 0, 2)).reshape(E * D_IN, D_FF)
    co = _dot(Wo_ref[...].reshape(E * D_FF, D_HID), Wp_ref[...])
    co_rows = jnp.transpose(co.reshape(E, D_FF, D_IN),
                            (0, 2, 1)).reshape(E * D_IN, D_FF)
    bi_eff = _dot(wi2, be_ref[...], _TN).reshape(E, D_FF) + bi_ref[...]
    bo_p = _dot(bo_ref[...], Wp_ref[...])
    for half in (0, 1):
        _moe_half(x_ref, We_ref, be_ref, Wg_ref, ci_rows, bi_eff, co_rows,
                  bo_p, bp_ref, tile4_ref, g4_ref, s32_ref, prefix_ref,
                  out_ref, half)


def _moe_half(x_ref, We_ref, be_ref, Wg_ref, ci_rows, bi_eff, co_rows, bo_p,
              bp_ref, tile4_ref, g4_ref, s32_ref, prefix_ref, out_ref, half):
    f32 = jnp.float32
    bf16 = jnp.bfloat16
    HH = TT // 2
    x = x_ref[pl.ds(half * HH, HH), :]
    h = _dot(We_ref[...], x, _TX) + be_ref[...]
    logits = _dot(Wg_ref[...], h, _TN)
    m = jnp.max(logits, axis=0, keepdims=True)
    el = jnp.exp(logits - m)
    gate = 1.0 / jnp.sum(el, axis=0, keepdims=True)
    is_max = (logits >= m).astype(bf16)
    prior = _dot(prefix_ref[...], is_max)
    onehot = jnp.where(prior == 0.0, is_max, jnp.zeros_like(is_max))
    onehot_f = onehot.astype(f32)
    mask = _dot(s32_ref[...], onehot)
    x_exp = _dot(tile4_ref[...], x, _NT) * mask
    pre = _dot(ci_rows, x_exp, _TN) + _dot(bi_eff, onehot_f, _TN)
    mid = jax.nn.gelu(pre)
    o2 = _dot(co_rows, mid) * mask
    out_t = ((_dot(g4_ref[...], o2) + _dot(bo_p, onehot_f, _TN)) * gate
             + bp_ref[...])
    out_ref[pl.ds(half * HH, HH), :] = out_t.T


def kernel(x, W_embed, b_embed, W_gate, Wi, bi, Wo, bo, W_proj, b_proj):
    full = lambda a: pl.BlockSpec(a.shape, lambda i: (0,) * a.ndim)
    args = (W_embed, b_embed.reshape(D_HID, 1), W_gate, Wi, bi, Wo, bo,
            W_proj, b_proj.reshape(D_IN, 1),
            jnp.asarray(_TILE4), jnp.asarray(_G4),
            jnp.asarray(_S32, dtype=jnp.bfloat16),
            jnp.asarray(_PREFIX_T, dtype=jnp.bfloat16))
    return pl.pallas_call(
        _moe_block,
        grid=(T // TT,),
        in_specs=[pl.BlockSpec((TT, D_IN), lambda i: (i, 0))]
                 + [full(a) for a in args],
        out_specs=pl.BlockSpec((TT, D_IN), lambda i: (i, 0)),
        out_shape=jax.ShapeDtypeStruct((T, D_IN), jnp.float32),
        compiler_params=pltpu.CompilerParams(
            dimension_semantics=("parallel",)),
    )(x, *args)


# speedup vs baseline: 1.0803x; 1.0056x over previous
import jax
import jax.numpy as jnp
import numpy as np
from jax import lax
from jax.experimental import pallas as pl
from jax.experimental.pallas import tpu as pltpu

T = 32768
D_IN = 4
D_HID = 16
D_FF = 32
E = 8
TT = 8192
_PARTS = 1

_EYE_E = np.eye(E, dtype=np.float32)
_TILE4 = np.tile(np.eye(D_IN, dtype=np.float32), (E, 1))
_G4 = np.tile(np.eye(D_IN, dtype=np.float32), (1, E))
_S32 = np.repeat(_EYE_E, D_IN, axis=0)
_PREFIX_T = np.tril(np.ones((E, E), dtype=np.float32), -1)

_NN = (((1,), (0,)), ((), ()))
_TN = (((0,), (0,)), ((), ()))
_TX = (((0,), (1,)), ((), ()))
_NT = (((1,), (1,)), ((), ()))


def _dot(a, b, dn=_NN):
    return lax.dot_general(a, b, dn, preferred_element_type=jnp.float32)


def _moe_block(x_ref, We_ref, be_ref, Wg_ref, Wi_ref, bi_ref, Wo_ref, bo_ref,
               Wp_ref, bp_ref, tile4_ref, g4_ref, s32_ref, prefix_ref,
               out_ref):
    wi2 = jnp.transpose(Wi_ref[...], (1, 0, 2)).reshape(D_HID, E * D_FF)
    ci = _dot(We_ref[...], wi2)
    ci_rows = jnp.transpose(ci.reshape(D_IN, E, D_FF),
                            (1, 0, 2)).reshape(E * D_IN, D_FF)
    co = _dot(Wo_ref[...].reshape(E * D_FF, D_HID), Wp_ref[...])
    co_rows = jnp.transpose(co.reshape(E, D_FF, D_IN),
                            (0, 2, 1)).reshape(E * D_IN, D_FF)
    bi_eff = _dot(wi2, be_ref[...], _TN).reshape(E, D_FF) + bi_ref[...]
    bo_p = _dot(bo_ref[...], Wp_ref[...])
    for part in range(_PARTS):
        _moe_half(x_ref, We_ref, be_ref, Wg_ref, ci_rows, bi_eff, co_rows,
                  bo_p, bp_ref, tile4_ref, g4_ref, s32_ref, prefix_ref,
                  out_ref, part)


def _moe_half(x_ref, We_ref, be_ref, Wg_ref, ci_rows, bi_eff, co_rows, bo_p,
              bp_ref, tile4_ref, g4_ref, s32_ref, prefix_ref, out_ref, half):
    f32 = jnp.float32
    bf16 = jnp.bfloat16
    HH = TT // _PARTS
    x = x_ref[pl.ds(half * HH, HH), :]
    h = _dot(We_ref[...], x, _TX) + be_ref[...]
    logits = _dot(Wg_ref[...], h, _TN)
    m = jnp.max(logits, axis=0, keepdims=True)
    el = jnp.exp(logits - m)
    gate = 1.0 / jnp.sum(el, axis=0, keepdims=True)
    is_max = (logits >= m).astype(bf16)
    prior = _dot(prefix_ref[...], is_max)
    onehot = jnp.where(prior == 0.0, is_max, jnp.zeros_like(is_max))
    onehot_f = onehot.astype(f32)
    mask = _dot(s32_ref[...], onehot)
    x_exp = _dot(tile4_ref[...], x, _NT) * mask
    pre = _dot(ci_rows, x_exp, _TN) + _dot(bi_eff, onehot_f, _TN)
    mid = jax.nn.gelu(pre)
    o2 = _dot(co_rows, mid) * mask
    out_t = ((_dot(g4_ref[...], o2) + _dot(bo_p, onehot_f, _TN)) * gate
             + bp_ref[...])
    out_ref[pl.ds(half * HH, HH), :] = out_t.T


def kernel(x, W_embed, b_embed, W_gate, Wi, bi, Wo, bo, W_proj, b_proj):
    full = lambda a: pl.BlockSpec(a.shape, lambda i: (0,) * a.ndim)
    args = (W_embed, b_embed.reshape(D_HID, 1), W_gate, Wi, bi, Wo, bo,
            W_proj, b_proj.reshape(D_IN, 1),
            jnp.asarray(_TILE4), jnp.asarray(_G4),
            jnp.asarray(_S32, dtype=jnp.bfloat16),
            jnp.asarray(_PREFIX_T, dtype=jnp.bfloat16))
    return pl.pallas_call(
        _moe_block,
        grid=(T // TT,),
        in_specs=[pl.BlockSpec((TT, D_IN), lambda i: (i, 0))]
                 + [full(a) for a in args],
        out_specs=pl.BlockSpec((TT, D_IN), lambda i: (i, 0)),
        out_shape=jax.ShapeDtypeStruct((T, D_IN), jnp.float32),
        compiler_params=pltpu.CompilerParams(
            dimension_semantics=("parallel",)),
    )(x, *args)
